# Initial kernel scaffold; baseline (speedup 1.0000x reference)
import jax
import jax.numpy as jnp
from jax.experimental import pallas as pl


def kernel(features, edge_index, bn_gamma, bn_beta, W1, b1, W2, b2, W3, b3):
    raise NotImplementedError("write your pallas kernel here")


# speedup vs baseline: 36.8933x; 36.8933x over previous
import functools

import jax
import jax.numpy as jnp
from jax import lax
from jax.experimental import pallas as pl
from jax.experimental.pallas import tpu as pltpu
from jax.experimental.pallas import tpu_sc as plsc

N_NODES = 100000
N_EDGES = 3200000

NC = 2
NS = 16

W = 128
K1 = 16
K2 = 4

E_PAD = ((N_EDGES + NC * NS * W * K1 - 1) // (NC * NS * W * K1)) * (NC * NS * W * K1)
WN = E_PAD // W
WIN_PER_WORKER = WN // (NC * NS)
G1 = WIN_PER_WORKER // K1
WIN_PER_SUB = WN // NS
G2 = WIN_PER_SUB // K2
assert WIN_PER_WORKER % K1 == 0 and WIN_PER_SUB % K2 == 0

NP = ((N_NODES + 8 + NS * 8 - 1) // (NS * 8)) * (NS * 8)
RPW = NP // NS

_MESH = plsc.VectorSubcoreMesh(core_axis_name="c", subcore_axis_name="s")


def _make_group_ops(x_hbm, src_hbm, dst_hbm, acc, sem_g, sem_s, K):
    def load_idx(win0, src_v, dst_v):
        pltpu.sync_copy(src_hbm.at[pl.ds(win0, K)], src_v)
        pltpu.sync_copy(dst_hbm.at[pl.ds(win0, K)], dst_v)

    def fire_gathers(src_v, rows):
        for j in range(K):
            pltpu.async_copy(x_hbm.at[src_v.at[j]], rows.at[j], sem_g)

    def wait_gathers(src_v, rows):
        for j in range(K):
            pltpu.make_async_copy(x_hbm.at[src_v.at[j]], rows.at[j],
                                  sem_g).wait()

    def fire_scatters(dst_v, rows):
        for j in range(K):
            pltpu.async_copy(rows.at[j], acc.at[dst_v.at[j]], sem_s, add=True)

    def wait_scatters(dst_v, rows):
        for j in range(K):
            pltpu.make_async_copy(rows.at[j], acc.at[dst_v.at[j]],
                                  sem_s).wait()

    return load_idx, fire_gathers, wait_gathers, fire_scatters, wait_scatters


def _pipelined_agg(x_hbm, src_hbm, dst_hbm, z_hbm, acc, bufs, sem_g, sem_s,
                   K, G, win_base, sub, writeback):
    load_idx, fire_g, wait_g, fire_s, wait_s = _make_group_ops(
        x_hbm, src_hbm, dst_hbm, acc, sem_g, sem_s, K)

    load_idx(win_base, *bufs[0][:2])
    fire_g(bufs[0][0], bufs[0][2])
    pltpu.sync_copy(z_hbm.at[pl.ds(sub * RPW, RPW)],
                    acc.at[pl.ds(sub * RPW, RPW)])
    plsc.subcore_barrier()

    def half(g, p, prefetch=True):
        sv, dv, rows = bufs[p]
        svn, dvn, rowsn = bufs[1 - p]

        if prefetch:
            @pl.when(g + 1 < G)
            def _():
                @pl.when(g >= 1)
                def _():
                    wait_s(dvn, rowsn)
                load_idx(win_base + (g + 1) * K, svn, dvn)
                fire_g(svn, rowsn)
        wait_g(sv, rows)
        fire_s(dv, rows)

    def pair(i, carry):
        half(2 * i, 0)
        half(2 * i + 1, 1)
        return carry

    lax.fori_loop(0, G // 2, pair, 0)
    if G % 2:
        half(G - 1, 0, prefetch=False)

    wait_s(bufs[0][1], bufs[0][2])
    wait_s(bufs[1][1], bufs[1][2])
    plsc.subcore_barrier()
    writeback()


def _agg_edge_split(xp, src2d, dst2d, zeros8):
    @functools.partial(
        pl.kernel,
        out_type=jax.ShapeDtypeStruct((NC, NP, 8), jnp.float32),
        mesh=_MESH,
        compiler_params=pltpu.CompilerParams(use_tc_tiling_on_sc=False),
        scratch_types=[
            pltpu.VMEM((K1, W), jnp.int32),
            pltpu.VMEM((K1, W), jnp.int32),
            pltpu.VMEM((K1, W, 8), jnp.float32),
            pltpu.VMEM((K1, W), jnp.int32),
            pltpu.VMEM((K1, W), jnp.int32),
            pltpu.VMEM((K1, W, 8), jnp.float32),
            pltpu.VMEM_SHARED((NP, 8), jnp.float32),
            pltpu.SemaphoreType.DMA,
            pltpu.SemaphoreType.DMA,
        ],
    )
    def k(xp_hbm, src_hbm, dst_hbm, z_hbm, out_hbm,
          sv0, dv0, rows0, sv1, dv1, rows1, acc, sem_g, sem_s):
        c = lax.axis_index("c")
        s = lax.axis_index("s")
        w = c * NS + s

        def writeback():
            pltpu.sync_copy(acc.at[pl.ds(s * RPW, RPW)],
                            out_hbm.at[c].at[pl.ds(s * RPW, RPW)])

        _pipelined_agg(xp_hbm, src_hbm, dst_hbm, z_hbm, acc,
                       ((sv0, dv0, rows0), (sv1, dv1, rows1)),
                       sem_g, sem_s, K1, G1, w * WIN_PER_WORKER, s, writeback)

    return k(xp, src2d, dst2d, zeros8)


def _agg_col_split(xlo, xhi, src2d, dst2d, zeros16):
    @functools.partial(
        pl.kernel,
        out_type=(
            jax.ShapeDtypeStruct((NP, 16), jnp.float32),
            jax.ShapeDtypeStruct((NP, 16), jnp.float32),
        ),
        mesh=_MESH,
        compiler_params=pltpu.CompilerParams(use_tc_tiling_on_sc=False),
        scratch_types=[
            pltpu.VMEM((K2, W), jnp.int32),
            pltpu.VMEM((K2, W), jnp.int32),
            pltpu.VMEM((K2, W, 16), jnp.float32),
            pltpu.VMEM((K2, W), jnp.int32),
            pltpu.VMEM((K2, W), jnp.int32),
            pltpu.VMEM((K2, W, 16), jnp.float32),
            pltpu.VMEM_SHARED((NP, 16), jnp.float32),
            pltpu.SemaphoreType.DMA,
            pltpu.SemaphoreType.DMA,
        ],
    )
    def k(xlo_hbm, xhi_hbm, src_hbm, dst_hbm, z_hbm, lo_out, hi_out,
          sv0, dv0, rows0, sv1, dv1, rows1, acc, sem_g, sem_s):
        c = lax.axis_index("c")
        s = lax.axis_index("s")
        bufs = ((sv0, dv0, rows0), (sv1, dv1, rows1))

        @pl.when(c == 0)
        def _():
            def writeback():
                pltpu.sync_copy(acc.at[pl.ds(s * RPW, RPW)],
                                lo_out.at[pl.ds(s * RPW, RPW)])
            _pipelined_agg(xlo_hbm, src_hbm, dst_hbm, z_hbm, acc, bufs,
                           sem_g, sem_s, K2, G2, s * WIN_PER_SUB, s, writeback)

        @pl.when(c == 1)
        def _():
            def writeback():
                pltpu.sync_copy(acc.at[pl.ds(s * RPW, RPW)],
                                hi_out.at[pl.ds(s * RPW, RPW)])
            _pipelined_agg(xhi_hbm, src_hbm, dst_hbm, z_hbm, acc, bufs,
                           sem_g, sem_s, K2, G2, s * WIN_PER_SUB, s, writeback)

    return k(xlo, xhi, src2d, dst2d, zeros16)


_TB = 2000
_NB = N_NODES // _TB


def _tc_stats_pad(feat):
    def body(feat_ref, featp_ref, stats_ref, acc):
        i = pl.program_id(0)
        x = feat_ref[...]
        b = x.shape[0]
        ones = jnp.ones((b, 1), jnp.float32)
        zeros = jnp.zeros((b, 3), jnp.float32)
        featp_ref[...] = jnp.concatenate([x, ones, zeros], axis=1)
        partial = jnp.stack([jnp.sum(x, axis=0), jnp.sum(x * x, axis=0)])

        @pl.when(i == 0)
        def _():
            acc[...] = partial

        @pl.when(i > 0)
        def _():
            acc[...] += partial

        @pl.when(i == _NB - 1)
        def _():
            stats_ref[...] = acc[...]

    return pl.pallas_call(
        body,
        grid=(_NB,),
        in_specs=[pl.BlockSpec((_TB, 4), lambda i: (i, 0))],
        out_specs=(
            pl.BlockSpec((_TB, 8), lambda i: (i, 0)),
            pl.BlockSpec((2, 4), lambda i: (0, 0)),
        ),
        out_shape=(
            jax.ShapeDtypeStruct((N_NODES, 8), jnp.float32),
            jax.ShapeDtypeStruct((2, 4), jnp.float32),
        ),
        scratch_shapes=[pltpu.VMEM((2, 4), jnp.float32)],
    )(feat)


def _tc_layer1(P, stats, gamma2d, beta2d, W1, b1_2d):
    def body(p_ref, stats_ref, g_ref, be_ref, w1_ref, b1_ref, lo_ref, hi_ref):
        A = p_ref[0] + p_ref[1]
        afeat = A[:, :4]
        deg = A[:, 4:5]
        sums = stats_ref[0, :]
        sumsq = stats_ref[1, :]
        n = jnp.float32(N_NODES)
        mean = sums / n
        var = sumsq / n - mean * mean
        s = g_ref[0, :] * lax.rsqrt(var + 1e-5)
        t = be_ref[0, :] - mean * s
        xb = afeat * s[None, :] + deg * t[None, :]
        y = jnp.dot(xb, w1_ref[...], preferred_element_type=jnp.float32)
        y = jnp.maximum(y + b1_ref[0, :], 0.0)
        lo_ref[...] = y[:, :16]
        hi_ref[...] = y[:, 16:]

    return pl.pallas_call(
        body,
        grid=(_NB,),
        in_specs=[
            pl.BlockSpec((2, _TB, 8), lambda i: (0, i, 0)),
            pl.BlockSpec((2, 4), lambda i: (0, 0)),
            pl.BlockSpec((1, 4), lambda i: (0, 0)),
            pl.BlockSpec((1, 4), lambda i: (0, 0)),
            pl.BlockSpec((4, 32), lambda i: (0, 0)),
            pl.BlockSpec((1, 32), lambda i: (0, 0)),
        ],
        out_specs=(
            pl.BlockSpec((_TB, 16), lambda i: (i, 0)),
            pl.BlockSpec((_TB, 16), lambda i: (i, 0)),
        ),
        out_shape=(
            jax.ShapeDtypeStruct((N_NODES, 16), jnp.float32),
            jax.ShapeDtypeStruct((N_NODES, 16), jnp.float32),
        ),
    )(P, stats, gamma2d, beta2d, W1, b1_2d)


def _tc_layer2(Alo, Ahi, W2, b2_2d, W3):
    def body(lo_ref, hi_ref, w2_ref, b2_ref, w3_ref, out_ref):
        y = (jnp.dot(lo_ref[...], w2_ref[:16, :],
                     preferred_element_type=jnp.float32)
             + jnp.dot(hi_ref[...], w2_ref[16:, :],
                       preferred_element_type=jnp.float32))
        y = jnp.maximum(y + b2_ref[0, :], 0.0)
        t2 = jnp.dot(y, w3_ref[...], preferred_element_type=jnp.float32)
        b = t2.shape[0]
        out_ref[...] = jnp.concatenate(
            [t2, jnp.zeros((b, 3), jnp.float32)], axis=1)

    return pl.pallas_call(
        body,
        grid=(_NB,),
        in_specs=[
            pl.BlockSpec((_TB, 16), lambda i: (i, 0)),
            pl.BlockSpec((_TB, 16), lambda i: (i, 0)),
            pl.BlockSpec((32, 32), lambda i: (0, 0)),
            pl.BlockSpec((1, 32), lambda i: (0, 0)),
            pl.BlockSpec((32, 5), lambda i: (0, 0)),
        ],
        out_specs=pl.BlockSpec((_TB, 8), lambda i: (i, 0)),
        out_shape=jax.ShapeDtypeStruct((N_NODES, 8), jnp.float32),
    )(Alo, Ahi, W2, b2_2d, W3)


def _tc_layer3(Q, b3_2d):
    def body(q_ref, b3_ref, out_ref):
        A = q_ref[0] + q_ref[1]
        out_ref[...] = A[:, :5] + b3_ref[0, :]

    return pl.pallas_call(
        body,
        grid=(_NB,),
        in_specs=[
            pl.BlockSpec((2, _TB, 8), lambda i: (0, i, 0)),
            pl.BlockSpec((1, 5), lambda i: (0, 0)),
        ],
        out_specs=pl.BlockSpec((_TB, 5), lambda i: (i, 0)),
        out_shape=jax.ShapeDtypeStruct((N_NODES, 5), jnp.float32),
    )(Q, b3_2d)


def kernel(features, edge_index, bn_gamma, bn_beta, W1, b1, W2, b2, W3, b3):
    src = edge_index[0].astype(jnp.int32)
    dst = edge_index[1].astype(jnp.int32)

    pad = E_PAD - N_EDGES
    pad_ar = jnp.arange(pad, dtype=jnp.int32)
    src_p = jnp.concatenate([src, pad_ar % 997])
    dst_p = jnp.concatenate([dst, N_NODES + (pad_ar % 8)])
    src2d = src_p.reshape(WN, W)
    dst2d = dst_p.reshape(WN, W)

    zeros8 = jnp.zeros((NP, 8), jnp.float32)
    zeros16 = jnp.zeros((NP, 16), jnp.float32)

    feat_p, stats = _tc_stats_pad(features)
    P = _agg_edge_split(feat_p, src2d, dst2d, zeros8)
    xlo, xhi = _tc_layer1(P, stats,
                          bn_gamma.reshape(1, 4), bn_beta.reshape(1, 4),
                          W1, b1.reshape(1, 32))
    Alo, Ahi = _agg_col_split(xlo, xhi, src2d, dst2d, zeros16)
    t2p = _tc_layer2(Alo, Ahi, W2, b2.reshape(1, 32), W3)
    Q = _agg_edge_split(t2p, src2d, dst2d, zeros8)
    return _tc_layer3(Q, b3.reshape(1, 5))
